# baseline (speedup 1.0000x reference)
import functools

import numpy as np
import jax
import jax.numpy as jnp
from jax import lax
from jax.experimental import pallas as pl
from jax.experimental.pallas import tpu as pltpu
from jax.experimental.pallas import tpu_sc as plsc

_MASK_ID = 32000
_T = 10
_B, _S = 4, 8192

_NC, _NS = 2, 16
_NW = _NC * _NS
_CHUNK = _B * _S // _NW
_CPR = _S // _CHUNK
_V = 16


def _np_mask_tables():
    ks = (np.uint32(0), np.uint32(42), np.uint32(0x1BD11BDA ^ 42))
    rot = ((13, 15, 26, 6), (17, 29, 16, 24))
    n = np.arange(_B * _S, dtype=np.uint32)
    x0 = np.full_like(n, ks[0])
    x1 = n + ks[1]
    for i in range(5):
        for r in rot[i % 2]:
            x0 = (x0 + x1).astype(np.uint32)
            x1 = (((x1 << np.uint32(r)) | (x1 >> np.uint32(32 - r))) ^ x0).astype(
                np.uint32
            )
        x0 = (x0 + ks[(i + 1) % 3]).astype(np.uint32)
        x1 = (x1 + ks[(i + 2) % 3] + np.uint32(i + 1)).astype(np.uint32)
    mant = ((x0 ^ x1) >> np.uint32(9)).astype(np.int32).reshape(_B, _S)

    p = (np.arange(_T, dtype=np.float64) * 0.1).astype(np.float32)
    thr = np.ceil(p.astype(np.float64) * 2.0**23).astype(np.int32)

    mb = np.zeros((_B * _T, _S), np.bool_)
    for i in range(_B):
        for t in range(_T):
            mb[i * _T + t] = mant[i] < thr[t]
    return mb.astype(np.int32), mb


_MI32, _MB = _np_mask_tables()

_mesh = plsc.VectorSubcoreMesh(core_axis_name="c", subcore_axis_name="s")


@functools.partial(
    pl.kernel,
    out_type=(
        jax.ShapeDtypeStruct((_B, _S), jnp.int32),
        jax.ShapeDtypeStruct((_B, _S), jnp.int32),
        jax.ShapeDtypeStruct((_B, _S), jnp.bool_),
    ),
    mesh=_mesh,
    compiler_params=pltpu.CompilerParams(needs_layout_passes=False),
    scratch_types=[
        pltpu.VMEM((_V,), jnp.int32),
        pltpu.VMEM((_CHUNK,), jnp.int32),
        pltpu.VMEM((_CHUNK,), jnp.int32),
        pltpu.VMEM((_CHUNK,), jnp.int32),
        pltpu.VMEM((_CHUNK,), jnp.int32),
    ],
)
def _sc_mask(
    t_hbm,
    tokens_hbm,
    mi_hbm,
    mb_hbm,
    noisy_hbm,
    labels_hbm,
    mask_hbm,
    tvm,
    civ,
    tokv,
    nozv,
    labv,
):
    wid = lax.axis_index("c") * _NS + lax.axis_index("s")
    row = wid // _CPR
    col0 = (wid % _CPR) * _CHUNK

    pltpu.sync_copy(t_hbm, tvm.at[pl.ds(0, _B)])
    lanes = lax.iota(jnp.int32, _V)
    tr = jnp.max(jnp.where(lanes == row, tvm[...], 0), axis=0)
    tr = jnp.clip(tr, 0, _T - 1)
    g = row * _T + tr

    pltpu.sync_copy(
        mb_hbm.at[g, pl.ds(col0, _CHUNK)], mask_hbm.at[row, pl.ds(col0, _CHUNK)]
    )

    pltpu.sync_copy(mi_hbm.at[g, pl.ds(col0, _CHUNK)], civ)
    pltpu.sync_copy(tokens_hbm.at[row, pl.ds(col0, _CHUNK)], tokv)
    for j in range(_CHUNK // _V):
        sl = pl.ds(j * _V, _V)
        v = tokv[sl]
        c = civ[sl] != 0
        nozv[sl] = jnp.where(c, jnp.int32(_MASK_ID), v)
        labv[sl] = jnp.where(c, v, jnp.int32(-100))
    pltpu.sync_copy(nozv, noisy_hbm.at[row, pl.ds(col0, _CHUNK)])
    pltpu.sync_copy(labv, labels_hbm.at[row, pl.ds(col0, _CHUNK)])


def kernel(tokens, t):
    noisy, labels, mask = _sc_mask(t, tokens, _MI32, _MB)
    return (noisy, labels, t, mask)


# speedup vs baseline: 1.0591x; 1.0591x over previous
import functools

import numpy as np
import jax
import jax.numpy as jnp
from jax import lax
from jax.experimental import pallas as pl
from jax.experimental.pallas import tpu as pltpu
from jax.experimental.pallas import tpu_sc as plsc

_MASK_ID = 32000
_T = 10
_B, _S = 4, 8192

_NC, _NS = 2, 16
_NW = _NC * _NS
_CHUNK = _B * _S // _NW
_CPR = _S // _CHUNK
_V = 16


def _np_mask_tables():
    ks = (np.uint32(0), np.uint32(42), np.uint32(0x1BD11BDA ^ 42))
    rot = ((13, 15, 26, 6), (17, 29, 16, 24))
    n = np.arange(_B * _S, dtype=np.uint32)
    x0 = np.full_like(n, ks[0])
    x1 = n + ks[1]
    for i in range(5):
        for r in rot[i % 2]:
            x0 = (x0 + x1).astype(np.uint32)
            x1 = (((x1 << np.uint32(r)) | (x1 >> np.uint32(32 - r))) ^ x0).astype(
                np.uint32
            )
        x0 = (x0 + ks[(i + 1) % 3]).astype(np.uint32)
        x1 = (x1 + ks[(i + 2) % 3] + np.uint32(i + 1)).astype(np.uint32)
    mant = ((x0 ^ x1) >> np.uint32(9)).astype(np.int32).reshape(_B, _S)

    p = (np.arange(_T, dtype=np.float64) * 0.1).astype(np.float32)
    thr = np.ceil(p.astype(np.float64) * 2.0**23).astype(np.int32)

    mb = np.zeros((_B * _T, _S), np.bool_)
    for i in range(_B):
        for t in range(_T):
            mb[i * _T + t] = mant[i] < thr[t]
    return mb.astype(np.int32), mb


_MI32, _MB = _np_mask_tables()

_mesh = plsc.VectorSubcoreMesh(core_axis_name="c", subcore_axis_name="s")


@functools.partial(
    pl.kernel,
    out_type=(
        jax.ShapeDtypeStruct((_B, _S), jnp.int32),
        jax.ShapeDtypeStruct((_B, _S), jnp.int32),
        jax.ShapeDtypeStruct((_B, _S), jnp.bool_),
    ),
    mesh=_mesh,
    compiler_params=pltpu.CompilerParams(needs_layout_passes=False),
    scratch_types=[
        pltpu.VMEM((_V,), jnp.int32),
        pltpu.VMEM((_CHUNK,), jnp.int32),
        pltpu.VMEM((_CHUNK,), jnp.int32),
        pltpu.VMEM((_CHUNK,), jnp.int32),
        pltpu.VMEM((_CHUNK,), jnp.int32),
        pltpu.SemaphoreType.DMA,
        pltpu.SemaphoreType.DMA,
    ],
)
def _sc_mask(
    t_hbm,
    tokens_hbm,
    mi_hbm,
    mb_hbm,
    noisy_hbm,
    labels_hbm,
    mask_hbm,
    tvm,
    civ,
    tokv,
    nozv,
    labv,
    sem_in,
    sem_out,
):
    wid = lax.axis_index("c") * _NS + lax.axis_index("s")
    row = wid // _CPR
    col0 = (wid % _CPR) * _CHUNK

    cp_tok = pltpu.async_copy(
        tokens_hbm.at[row, pl.ds(col0, _CHUNK)], tokv, sem_in
    )
    cp_t = pltpu.async_copy(t_hbm, tvm.at[pl.ds(0, _B)], sem_out)
    cp_t.wait()
    lanes = lax.iota(jnp.int32, _V)
    tr = jnp.max(jnp.where(lanes == row, tvm[...], 0), axis=0)
    tr = jnp.clip(tr, 0, _T - 1)
    g = row * _T + tr

    cp_mask = pltpu.async_copy(
        mb_hbm.at[g, pl.ds(col0, _CHUNK)],
        mask_hbm.at[row, pl.ds(col0, _CHUNK)],
        sem_out,
    )
    cp_ci = pltpu.async_copy(mi_hbm.at[g, pl.ds(col0, _CHUNK)], civ, sem_in)
    cp_tok.wait()
    cp_ci.wait()

    for j in range(_CHUNK // _V):
        sl = pl.ds(j * _V, _V)
        v = tokv[sl]
        c = civ[sl] != 0
        nozv[sl] = jnp.where(c, jnp.int32(_MASK_ID), v)
        labv[sl] = jnp.where(c, v, jnp.int32(-100))
    cp_noz = pltpu.async_copy(nozv, noisy_hbm.at[row, pl.ds(col0, _CHUNK)], sem_in)
    cp_lab = pltpu.async_copy(labv, labels_hbm.at[row, pl.ds(col0, _CHUNK)], sem_in)
    cp_mask.wait()
    cp_noz.wait()
    cp_lab.wait()


def kernel(tokens, t):
    noisy, labels, mask = _sc_mask(t, tokens, _MI32, _MB)
    return (noisy, labels, t, mask)


# speedup vs baseline: 4.8111x; 4.5427x over previous
import numpy as np
import jax
import jax.numpy as jnp
from jax.experimental import pallas as pl
from jax.experimental.pallas import tpu as pltpu

_MASK_ID = 32000
_TIMESTEPS = 10
_B, _S = 4, 8192

_P_F32 = np.arange(_TIMESTEPS, dtype=np.float64) * 0.1
_THR = np.ceil(_P_F32.astype(np.float32).astype(np.float64) * 2.0**23).astype(
    np.int32
)

_KS = (np.uint32(0), np.uint32(42), np.uint32(0x1BD11BDA ^ 42))
_ROT = ((13, 15, 26, 6), (17, 29, 16, 24))


def _mask_kernel(t_ref, tokens_ref, noisy_ref, labels_ref, mask_ref):
    rows = jax.lax.broadcasted_iota(jnp.uint32, (_B, _S), 0)
    cols = jax.lax.broadcasted_iota(jnp.uint32, (_B, _S), 1)
    n = rows * jnp.uint32(_S) + cols

    x0 = jnp.full((_B, _S), _KS[0], jnp.uint32)
    x1 = n + _KS[1]
    for i in range(5):
        for r in _ROT[i % 2]:
            x0 = x0 + x1
            x1 = ((x1 << r) | (x1 >> (32 - r))) ^ x0
        x0 = x0 + _KS[(i + 1) % 3]
        x1 = x1 + _KS[(i + 2) % 3] + jnp.uint32(i + 1)
    mant = ((x0 ^ x1) >> 9).astype(jnp.int32)

    batch_row = jax.lax.broadcasted_iota(jnp.int32, (_B, 1), 0)
    thr = jnp.zeros((_B, 1), jnp.int32)
    for i in range(_B):
        ti = jnp.clip(t_ref[i], 0, _TIMESTEPS - 1)
        thr_i = jnp.int32(_THR[_TIMESTEPS - 1])
        for k in range(_TIMESTEPS - 1):
            thr_i = jnp.where(ti == k, jnp.int32(_THR[k]), thr_i)
        thr = jnp.where(batch_row == i, thr_i, thr)

    mask = mant < thr
    tokens = tokens_ref[...]
    noisy_ref[...] = jnp.where(mask, jnp.int32(_MASK_ID), tokens)
    labels_ref[...] = jnp.where(mask, tokens, jnp.int32(-100))
    mask_ref[...] = mask


def kernel(tokens, t):
    noisy, labels, mask = pl.pallas_call(
        _mask_kernel,
        compiler_params=pltpu.CompilerParams(
            skip_device_barrier=True,
            disable_bounds_checks=True,
            disable_semaphore_checks=True,
        ),
        in_specs=[
            pl.BlockSpec(memory_space=pltpu.SMEM),
            pl.BlockSpec(memory_space=pltpu.VMEM),
        ],
        out_shape=(
            jax.ShapeDtypeStruct((_B, _S), jnp.int32),
            jax.ShapeDtypeStruct((_B, _S), jnp.int32),
            jax.ShapeDtypeStruct((_B, _S), jnp.bool_),
        ),
    )(t, tokens)
    return (noisy, labels, t, mask)
